# baseline (device time: 48968 ns/iter reference)
import jax
import jax.numpy as jnp
from jax import lax
from jax.experimental import pallas as pl
from jax.experimental.pallas import tpu as pltpu

N_DEV = 16
SIZES = (256, 128, 64, 32)


def _gelu(z):
    return 0.5 * z * (1.0 + jnp.tanh(0.7978845608 * (z + 0.044715 * z * z * z)))


def kernel(A, B):
    m, k = A.shape
    _, n = B.shape

    def body(a_ref, b_ref, out_ref, z_ref,
             sa0, sa1, sa2, sa3, ra0, ra1, ra2, ra3,
             sb0, sb1, sb2, sb3, rb0, rb1, rb2, rb3,
             rsa_ssem, rsa_rsem, rsb_ssem, rsb_rsem,
             aga_ssem, aga_rsem, agb_ssem, agb_rsem):
        my = lax.axis_index("i")
        zc = my >> 2
        p = my & 3

        bx = (p ^ (p >> 1)) & 1
        by = p >> 1
        bz0 = zc & 1
        bz1 = zc >> 1
        px = (zc << 2) | (p ^ 1)
        py = (zc << 2) | (p ^ 3)
        pz0 = ((zc ^ 1) << 2) | p
        pz1 = ((zc ^ 2) << 2) | p

        bits_a, parts_a = [bx, by, bz0, bz1], [px, py, pz0, pz1]
        bits_b, parts_b = [by, bx, bz1, bz0], [py, px, pz1, pz0]

        barrier_sem = pltpu.get_barrier_semaphore()
        for nbr in parts_a:
            pl.semaphore_signal(barrier_sem, inc=1, device_id=(nbr,),
                                device_id_type=pl.DeviceIdType.MESH)

        b_bf = b_ref[...].astype(jnp.bfloat16)
        sa = [sa0, sa1, sa2, sa3]
        ra = [ra0, ra1, ra2, ra3]
        sb = [sb0, sb1, sb2, sb3]
        rb = [rb0, rb1, rb2, rb3]
        rdmas = []

        def exchange(src, dst, ssem, rsem, kk, partner):
            rdma = pltpu.make_async_remote_copy(
                src_ref=src, dst_ref=dst,
                send_sem=ssem.at[kk], recv_sem=rsem.at[kk],
                device_id=(partner,), device_id_type=pl.DeviceIdType.MESH,
            )
            rdma.start()
            rdmas.append(rdma)
            return rdma

        send_a = (1 - bits_a[0]) * 256
        start_a = bits_a[0] * 256
        send_b = 512 + (1 - bits_b[0]) * 256
        start_b = 512 + bits_b[0] * 256
        va = jnp.dot(a_ref[pl.ds(send_a, 256), :].astype(jnp.bfloat16),
                     b_bf, preferred_element_type=jnp.float32)
        z_ref[pl.ds(send_a, 256), :] = va
        sa[0][...] = va.astype(jnp.bfloat16)
        vb = jnp.dot(a_ref[pl.ds(send_b, 256), :].astype(jnp.bfloat16),
                     b_bf, preferred_element_type=jnp.float32)
        z_ref[pl.ds(send_b, 256), :] = vb
        sb[0][...] = vb.astype(jnp.bfloat16)
        pl.semaphore_wait(barrier_sem, 4)
        da = exchange(sa[0], ra[0], rsa_ssem, rsa_rsem, 0, parts_a[0])
        db = exchange(sb[0], rb[0], rsb_ssem, rsb_rsem, 0, parts_b[0])
        for s in (start_a, start_b):
            z_ref[pl.ds(s, 256), :] = jnp.dot(
                a_ref[pl.ds(s, 256), :].astype(jnp.bfloat16),
                b_bf, preferred_element_type=jnp.float32)

        for kk in range(1, 4):
            half = SIZES[kk]
            ra_s = (1 - bits_a[kk]) * half
            ra_k = bits_a[kk] * half
            rb_s = (1 - bits_b[kk]) * half
            rb_k = bits_b[kk] * half

            def fwd_a(kk=kk, half=half, ra_s=ra_s, start=start_a):
                da.wait_recv()
                v = (z_ref[pl.ds(start + ra_s, half), :]
                     + ra[kk - 1][pl.ds(ra_s, half), :].astype(jnp.float32))
                z_ref[pl.ds(start + ra_s, half), :] = v
                sa[kk][...] = v.astype(jnp.bfloat16)
                return exchange(sa[kk], ra[kk], rsa_ssem, rsa_rsem, kk, parts_a[kk])

            def fwd_b(kk=kk, half=half, rb_s=rb_s, start=start_b):
                db.wait_recv()
                v = (z_ref[pl.ds(start + rb_s, half), :]
                     + rb[kk - 1][pl.ds(rb_s, half), :].astype(jnp.float32))
                z_ref[pl.ds(start + rb_s, half), :] = v
                sb[kk][...] = v.astype(jnp.bfloat16)
                return exchange(sb[kk], rb[kk], rsb_ssem, rsb_rsem, kk, parts_b[kk])

            if kk % 2 == 1:
                da = fwd_a()
                db = fwd_b()
            else:
                db = fwd_b()
                da = fwd_a()

            z_ref[pl.ds(start_a + ra_k, half), :] = (
                z_ref[pl.ds(start_a + ra_k, half), :]
                + ra[kk - 1][pl.ds(ra_k, half), :].astype(jnp.float32))
            z_ref[pl.ds(start_b + rb_k, half), :] = (
                z_ref[pl.ds(start_b + rb_k, half), :]
                + rb[kk - 1][pl.ds(rb_k, half), :].astype(jnp.float32))
            start_a = start_a + ra_k
            start_b = start_b + rb_k

        da.wait_recv()
        ga = _gelu(z_ref[pl.ds(start_a, 32), :] + ra[3][...].astype(jnp.float32))
        base_a = start_a >> 5
        out_ref[pl.ds(base_a, 1)] = ga.astype(jnp.bfloat16)[None]
        da = exchange(out_ref.at[pl.ds(base_a, 1)], out_ref.at[pl.ds(base_a, 1)],
                      aga_ssem, aga_rsem, 0, parts_a[3])
        db.wait_recv()
        gb = _gelu(z_ref[pl.ds(start_b, 32), :] + rb[3][...].astype(jnp.float32))
        base_b = start_b >> 5
        out_ref[pl.ds(base_b, 1)] = gb.astype(jnp.bfloat16)[None]
        db = exchange(out_ref.at[pl.ds(base_b, 1)], out_ref.at[pl.ds(base_b, 1)],
                      agb_ssem, agb_rsem, 0, parts_b[3])

        for kk in range(1, 4):
            nch = 1 << kk
            da.wait_recv()
            base_a = base_a & ~(nch >> 1)
            da = exchange(out_ref.at[pl.ds(base_a, nch)],
                          out_ref.at[pl.ds(base_a, nch)],
                          aga_ssem, aga_rsem, kk, parts_a[3 - kk])
            db.wait_recv()
            base_b = base_b & ~(nch >> 1)
            db = exchange(out_ref.at[pl.ds(base_b, nch)],
                          out_ref.at[pl.ds(base_b, nch)],
                          agb_ssem, agb_rsem, kk, parts_b[3 - kk])
        da.wait_recv()
        db.wait_recv()

        for rdma in rdmas:
            rdma.wait_send()

    out = pl.pallas_call(
        body,
        out_shape=jax.ShapeDtypeStruct((32, m // 32, n), jnp.bfloat16),
        in_specs=[pl.BlockSpec(memory_space=pltpu.VMEM),
                  pl.BlockSpec(memory_space=pltpu.VMEM)],
        out_specs=pl.BlockSpec(memory_space=pltpu.VMEM),
        scratch_shapes=[
            pltpu.VMEM((m, n), jnp.float32),
            pltpu.VMEM((256, n), jnp.bfloat16),
            pltpu.VMEM((128, n), jnp.bfloat16),
            pltpu.VMEM((64, n), jnp.bfloat16),
            pltpu.VMEM((32, n), jnp.bfloat16),
            pltpu.VMEM((256, n), jnp.bfloat16),
            pltpu.VMEM((128, n), jnp.bfloat16),
            pltpu.VMEM((64, n), jnp.bfloat16),
            pltpu.VMEM((32, n), jnp.bfloat16),
            pltpu.VMEM((256, n), jnp.bfloat16),
            pltpu.VMEM((128, n), jnp.bfloat16),
            pltpu.VMEM((64, n), jnp.bfloat16),
            pltpu.VMEM((32, n), jnp.bfloat16),
            pltpu.VMEM((256, n), jnp.bfloat16),
            pltpu.VMEM((128, n), jnp.bfloat16),
            pltpu.VMEM((64, n), jnp.bfloat16),
            pltpu.VMEM((32, n), jnp.bfloat16),
            pltpu.SemaphoreType.DMA((4,)),
            pltpu.SemaphoreType.DMA((4,)),
            pltpu.SemaphoreType.DMA((4,)),
            pltpu.SemaphoreType.DMA((4,)),
            pltpu.SemaphoreType.DMA((4,)),
            pltpu.SemaphoreType.DMA((4,)),
            pltpu.SemaphoreType.DMA((4,)),
            pltpu.SemaphoreType.DMA((4,)),
        ],
        compiler_params=pltpu.CompilerParams(collective_id=0),
    )(A, B)
    return out.reshape(m, n)
